# baseline (device time: 12431 ns/iter reference)
import os

import jax
import jax.numpy as jnp
from jax import lax
from jax.experimental import pallas as pl
from jax.experimental.pallas import tpu as pltpu

_NOCOMM = os.environ.get("NOCOMM") == "1"
_NOCOMPUTE = os.environ.get("NOCOMPUTE") == "1"

N_DEV = 8
B, SQ, SKV, DH = 2, 128, 128, 64
H_LOC = 4
D_MODEL = 512
D_HID = H_LOC * DH
ROWS = B * SQ
CHUNK = ROWS // N_DEV


def kernel(x, Wq, K_ext, V_ext, Wo):
    def body(x_ref, wq_hbm, k_ref, v_ref, wo_hbm, out_ref,
             wq_ref, wo_ref, ctx_ref, rs_ref, ag_ref,
             send_sems, recv_sems, copy_sems):
        my = lax.axis_index("i")

        wq_copy = pltpu.make_async_copy(
            wq_hbm.at[:, pl.ds(my * D_HID, D_HID)], wq_ref, copy_sems.at[0])
        wq_copy.start()
        wo_copy = pltpu.make_async_copy(wo_hbm, wo_ref, copy_sems.at[1])
        wo_copy.start()

        if not _NOCOMM:
            barrier = pltpu.get_barrier_semaphore()
            for d in range(N_DEV):
                @pl.when(d != my)
                def _():
                    pl.semaphore_signal(
                        barrier, inc=1,
                        device_id=(d,), device_id_type=pl.DeviceIdType.MESH,
                    )
            pl.semaphore_wait(barrier, N_DEV - 1)

        wq_copy.wait()

        def attn_batch(b):
            qb = jnp.dot(x_ref[b], wq_ref[:],
                         preferred_element_type=jnp.float32)
            heads = []
            for h in range(H_LOC):
                q = qb[:, h * DH:(h + 1) * DH]
                k = k_ref[b, :, h, :]
                v = v_ref[b, :, h, :]
                s = lax.dot_general(
                    q, k, (((1,), (1,)), ((), ())),
                    preferred_element_type=jnp.float32) * 0.125
                m = jnp.max(s, axis=-1, keepdims=True)
                w = jnp.exp(s - m)
                w = w / jnp.sum(w, axis=-1, keepdims=True)
                heads.append(jnp.dot(w, v, preferred_element_type=jnp.float32))
            return jnp.concatenate(heads, axis=1)

        chunks_per_b = SQ // CHUNK
        for b in range(B):
            if _NOCOMPUTE:
                ctx_ref[b * SQ:(b + 1) * SQ, :] = (
                    x_ref[b, :, 0:D_HID].astype(jnp.bfloat16))
            else:
                ctx_ref[b * SQ:(b + 1) * SQ, :] = (
                    attn_batch(b).astype(jnp.bfloat16))
            for d in range(b * chunks_per_b if not _NOCOMM else 0,
                           (b + 1) * chunks_per_b if not _NOCOMM else 0):
                @pl.when(d != my)
                def _():
                    pltpu.make_async_remote_copy(
                        src_ref=ctx_ref.at[pl.ds(d * CHUNK, CHUNK), :],
                        dst_ref=rs_ref.at[my],
                        send_sem=send_sems.at[d],
                        recv_sem=recv_sems.at[0],
                        device_id=(d,),
                        device_id_type=pl.DeviceIdType.MESH,
                    ).start()

        rs_ref[my] = ctx_ref[pl.ds(my * CHUNK, CHUNK), :]

        for d in range(N_DEV if not _NOCOMM else 0):
            @pl.when(d != my)
            def _():
                pltpu.make_async_remote_copy(
                    src_ref=ctx_ref.at[pl.ds(0, CHUNK), :],
                    dst_ref=rs_ref.at[d],
                    send_sem=send_sems.at[d],
                    recv_sem=recv_sems.at[0],
                    device_id=(d,),
                    device_id_type=pl.DeviceIdType.MESH,
                ).wait_recv()

        wo_copy.wait()

        if _NOCOMPUTE:
            acc = jnp.concatenate(
                [rs_ref[0], rs_ref[1]], axis=1).astype(jnp.float32)
        else:
            acc = jnp.dot(rs_ref[0][:].astype(jnp.float32),
                          wo_ref[0:D_HID, :],
                          preferred_element_type=jnp.float32)
            for d in range(1, N_DEV):
                acc = acc + jnp.dot(rs_ref[d][:].astype(jnp.float32),
                                    wo_ref[d * D_HID:(d + 1) * D_HID, :],
                                    preferred_element_type=jnp.float32)
        ag_ref[pl.ds(my * CHUNK, CHUNK), :] = acc.astype(jnp.bfloat16)

        for d in range(N_DEV if not _NOCOMM else 0):
            @pl.when(d != my)
            def _():
                pltpu.make_async_remote_copy(
                    src_ref=ag_ref.at[pl.ds(my * CHUNK, CHUNK), :],
                    dst_ref=ag_ref.at[pl.ds(my * CHUNK, CHUNK), :],
                    send_sem=send_sems.at[N_DEV + d],
                    recv_sem=recv_sems.at[1],
                    device_id=(d,),
                    device_id_type=pl.DeviceIdType.MESH,
                ).start()

        for d in range(N_DEV if not _NOCOMM else 0):
            @pl.when(d != my)
            def _():
                pltpu.make_async_remote_copy(
                    src_ref=ag_ref.at[pl.ds(my * CHUNK, CHUNK), :],
                    dst_ref=ag_ref.at[pl.ds(d * CHUNK, CHUNK), :],
                    send_sem=send_sems.at[N_DEV + d],
                    recv_sem=recv_sems.at[1],
                    device_id=(d,),
                    device_id_type=pl.DeviceIdType.MESH,
                ).wait_recv()

        out_ref[:, :] = ag_ref[:].astype(jnp.float32)

        for d in range(N_DEV if not _NOCOMM else 0):
            @pl.when(d != my)
            def _():
                pltpu.make_async_remote_copy(
                    src_ref=ctx_ref.at[pl.ds(d * CHUNK, CHUNK), :],
                    dst_ref=rs_ref.at[my],
                    send_sem=send_sems.at[d],
                    recv_sem=recv_sems.at[0],
                    device_id=(d,),
                    device_id_type=pl.DeviceIdType.MESH,
                ).wait_send()
                pltpu.make_async_remote_copy(
                    src_ref=ag_ref.at[pl.ds(my * CHUNK, CHUNK), :],
                    dst_ref=ag_ref.at[pl.ds(my * CHUNK, CHUNK), :],
                    send_sem=send_sems.at[N_DEV + d],
                    recv_sem=recv_sems.at[1],
                    device_id=(d,),
                    device_id_type=pl.DeviceIdType.MESH,
                ).wait_send()

    out_flat = pl.pallas_call(
        body,
        out_shape=jax.ShapeDtypeStruct((ROWS, D_MODEL), jnp.float32),
        in_specs=[
            pl.BlockSpec(memory_space=pltpu.VMEM),
            pl.BlockSpec(memory_space=pl.ANY),
            pl.BlockSpec(memory_space=pltpu.VMEM),
            pl.BlockSpec(memory_space=pltpu.VMEM),
            pl.BlockSpec(memory_space=pl.ANY),
        ],
        out_specs=pl.BlockSpec(memory_space=pltpu.VMEM),
        scratch_shapes=[
            pltpu.VMEM((D_MODEL, D_HID), jnp.float32),
            pltpu.VMEM((H_LOC * DH * N_DEV, D_MODEL), jnp.float32),
            pltpu.VMEM((ROWS, D_HID), jnp.bfloat16),
            pltpu.VMEM((N_DEV, CHUNK, D_HID), jnp.bfloat16),
            pltpu.VMEM((ROWS, D_MODEL), jnp.bfloat16),
            pltpu.SemaphoreType.DMA((2 * N_DEV,)),
            pltpu.SemaphoreType.DMA((2,)),
            pltpu.SemaphoreType.DMA((2,)),
        ],
        compiler_params=(None if _NOCOMM
                         else pltpu.CompilerParams(collective_id=0)),
    )(x, Wq, K_ext, V_ext, Wo)
    return out_flat.reshape(B, SQ, D_MODEL)
